# baseline (speedup 1.0000x reference)
import functools

import jax
import jax.numpy as jnp
from jax import lax
from jax.experimental import pallas as pl
from jax.experimental.pallas import tpu as pltpu
from jax.experimental.pallas import tpu_sc as plsc

_NC, _NS, _L = 2, 16, 16
_NW = _NC * _NS


@functools.lru_cache(maxsize=None)
def _make_sc_gather(V, D, B, CH, NBUF, dtype=jnp.float32):
    b_per_w = B // _NW
    assert B % _NW == 0 and b_per_w % CH == 0
    n_chunks = b_per_w // CH
    assert n_chunks % NBUF == 0
    n_groups = n_chunks // NBUF
    mesh = plsc.VectorSubcoreMesh(core_axis_name="c", subcore_axis_name="s")

    scratch = [pltpu.VMEM((n_chunks, CH), jnp.int32)]
    scratch += [pltpu.VMEM((CH, D), dtype) for _ in range(NBUF)]
    scratch += [pltpu.SemaphoreType.DMA for _ in range(2 * NBUF)]

    @functools.partial(
        pl.kernel,
        mesh=mesh,
        out_type=jax.ShapeDtypeStruct((B, D), dtype),
        compiler_params=pltpu.CompilerParams(use_tc_tiling_on_sc=False),
        scratch_types=scratch,
    )
    def gather_k(table_hbm, idx_hbm, out_hbm, idx_v, *bufs_sems):
        bufs = bufs_sems[:NBUF]
        gsem = bufs_sems[NBUF:2 * NBUF]
        wsem = bufs_sems[2 * NBUF:]
        wid = lax.axis_index("s") * _NC + lax.axis_index("c")
        base = wid * b_per_w
        pltpu.sync_copy(idx_hbm.at[pl.ds(wid * n_chunks, n_chunks)], idx_v)

        def _wb_wait(b):
            pltpu.make_async_copy(bufs[b], out_hbm.at[pl.ds(base, CH)], wsem[b]).wait()

        def body(g, carry):
            handles = []
            for b in range(NBUF):
                i = g * NBUF + b

                @pl.when(g > 0)
                def _wait_prev_wb():
                    _wb_wait(b)

                handles.append(pltpu.async_copy(table_hbm.at[idx_v.at[i]], bufs[b], gsem[b]))
            for b in range(NBUF):
                i = g * NBUF + b
                handles[b].wait()
                pltpu.async_copy(bufs[b], out_hbm.at[pl.ds(base + i * CH, CH)], wsem[b])
            return carry

        lax.fori_loop(0, n_groups, body, 0)
        for b in range(NBUF):
            _wb_wait(b)

    return gather_k


def _sc_gather(table, idx):
    V, D = table.shape
    R, K = idx.shape
    B = R * K
    CH = 64 if D > 128 else 128
    nch = B // _NW // CH
    NBUF = 4 if nch % 4 == 0 else (2 if nch % 2 == 0 else nch)
    k = _make_sc_gather(V, D, B, CH, NBUF, table.dtype.type)
    return k(table, idx.reshape(B // CH, CH)).reshape(R, K, D)


def _pad_rows(idx2d, rows_to):
    n = idx2d.shape[0]
    return jnp.pad(idx2d, ((0, rows_to - n), (0, 0)))


def _head_kernel(x_ref, w_ref, b_ref, o_ref):
    o_ref[...] = x_ref[...] @ w_ref[...] + b_ref[...]


def _head(x, w, b):
    n = x.shape[0]
    wp = jnp.pad(w, ((0, 0), (0, 128 - w.shape[1])))
    bp = jnp.pad(b, ((0, 128 - b.shape[0]),)).reshape(1, 128)
    out = pl.pallas_call(
        _head_kernel,
        out_shape=jax.ShapeDtypeStruct((n, 128), jnp.float32),
        grid=(n // 2000,),
        in_specs=[
            pl.BlockSpec((2000, x.shape[1]), lambda i: (i, 0)),
            pl.BlockSpec((x.shape[1], 128), lambda i: (0, 0)),
            pl.BlockSpec((1, 128), lambda i: (0, 0)),
        ],
        out_specs=pl.BlockSpec((2000, 128), lambda i: (i, 0)),
    )(x, wp, bp)
    return out[:, :w.shape[1]]


def _leaky(x):
    return jax.nn.leaky_relu(x, 0.1)


def _bn(x, g, b):
    m = jnp.mean(x, axis=0, keepdims=True)
    v = jnp.var(x, axis=0, keepdims=True)
    return g * (x - m) * lax.rsqrt(v + 1e-5) + b


def _kpconv(q_pts, feats, nb_pad, nq, kpts, w, sigma):
    sp = jnp.pad(q_pts, ((0, 0), (0, 13)))
    npts = _sc_gather(sp, nb_pad)[:nq, :, :3]
    rel = npts - q_pts[:, None, :]
    d = jnp.sqrt(jnp.sum((rel[:, :, None, :] - kpts[None, None, :, :]) ** 2, axis=-1) + 1e-12)
    infl = jnp.maximum(0.0, 1.0 - d / sigma)
    nf = _sc_gather(feats.astype(jnp.bfloat16), nb_pad)[:nq]
    wf = jnp.einsum('nkp,nkc->npc', infl, nf)
    return jnp.einsum('npc,pcd->nd', wf, w)


def _inv_res(src_feats, nb_pad, nq, p, strided, groups=8):
    x = _leaky(_bn(src_feats @ p['wd'], p['g1'], p['b1']))
    nx = _sc_gather(x, nb_pad)[:nq]
    N, K, mid = nx.shape
    ctx = jnp.mean(nx, axis=1)
    attn = jax.nn.softmax((ctx @ p['wg']).reshape(N, K, groups), axis=1)
    nxg = nx.reshape(N, K, groups, mid // groups)
    agg = jnp.sum(attn[..., None] * nxg, axis=1).reshape(N, mid)
    out = _bn(agg @ p['wu'], p['g2'], p['b2'])
    if strided:
        sc = jnp.max(_sc_gather(src_feats, nb_pad)[:nq], axis=1)
    else:
        sc = src_feats
    if 'wsc' in p:
        sc = sc @ p['wsc']
    return _leaky(out + sc)


def _unary(x, p):
    return _leaky(_bn(x @ p['w'], p['g'], p['b']))


def _upsample(feats, ups_pad, nq, dists):
    nf = _sc_gather(feats, ups_pad)[:nq]
    w = 1.0 / (dists + 1e-6)
    w = w / jnp.sum(w, axis=1, keepdims=True)
    return jnp.sum(nf * w[..., None], axis=1)


def kernel(features, points0, points1, neighbors0, neighbors1, pools0, upsamples0, up_distances0, params):
    nb0_pad = _pad_rows(neighbors0, 10240)
    nb1_pad = _pad_rows(neighbors1, 2560)
    pools_pad = _pad_rows(pools0, 2560)
    ups_pad = _pad_rows(upsamples0, 10240)

    f = _leaky(_bn(_kpconv(points0, features, nb0_pad, 10000, params['kpts'], params['kpw'], 1.0), params['kpg'], params['kpb']))
    f = _inv_res(f, nb0_pad, 10000, params['res1'], False)
    skip = f
    f = _inv_res(f, pools_pad, 2500, params['pool1'], True)
    f = _inv_res(f, nb1_pad, 2500, params['res2'], False)
    f = _upsample(f, ups_pad, 10000, up_distances0)
    f = jnp.concatenate([f, skip], axis=1)
    f = _unary(f, params['dec1'])
    f = _unary(f, params['head1'])
    return _head(f, params['head_w'], params['head_b'])


# speedup vs baseline: 1.0528x; 1.0528x over previous
import functools

import jax
import jax.numpy as jnp
from jax import lax
from jax.experimental import pallas as pl
from jax.experimental.pallas import tpu as pltpu
from jax.experimental.pallas import tpu_sc as plsc

_NC, _NS, _L = 2, 16, 16
_NW = _NC * _NS


@functools.lru_cache(maxsize=None)
def _make_sc_gather(V, D, B, CH, NBUF, dtype=jnp.float32):
    b_per_w = B // _NW
    assert B % _NW == 0 and b_per_w % CH == 0
    n_chunks = b_per_w // CH
    assert n_chunks % NBUF == 0
    n_groups = n_chunks // NBUF
    mesh = plsc.VectorSubcoreMesh(core_axis_name="c", subcore_axis_name="s")

    scratch = [pltpu.VMEM((n_chunks, CH), jnp.int32)]
    scratch += [pltpu.VMEM((CH, D), dtype) for _ in range(NBUF)]
    scratch += [pltpu.SemaphoreType.DMA for _ in range(2 * NBUF)]

    @functools.partial(
        pl.kernel,
        mesh=mesh,
        out_type=jax.ShapeDtypeStruct((B, D), dtype),
        compiler_params=pltpu.CompilerParams(use_tc_tiling_on_sc=False),
        scratch_types=scratch,
    )
    def gather_k(table_hbm, idx_hbm, out_hbm, idx_v, *bufs_sems):
        bufs = bufs_sems[:NBUF]
        gsem = bufs_sems[NBUF:2 * NBUF]
        wsem = bufs_sems[2 * NBUF:]
        wid = lax.axis_index("s") * _NC + lax.axis_index("c")
        base = wid * b_per_w
        pltpu.sync_copy(idx_hbm.at[pl.ds(wid * n_chunks, n_chunks)], idx_v)

        def _wb_wait(b):
            pltpu.make_async_copy(bufs[b], out_hbm.at[pl.ds(base, CH)], wsem[b]).wait()

        def body(g, carry):
            handles = []
            for b in range(NBUF):
                @pl.when(g > 0)
                def _wait_prev_wb():
                    _wb_wait(b)

                handles.append(pltpu.async_copy(
                    table_hbm.at[idx_v.at[g * NBUF + b]], bufs[b], gsem[b]))
            for b in range(NBUF):
                i = g * NBUF + b
                handles[b].wait()
                pltpu.async_copy(bufs[b], out_hbm.at[pl.ds(base + i * CH, CH)], wsem[b])
            return carry

        lax.fori_loop(0, n_groups, body, 0)
        for b in range(NBUF):
            _wb_wait(b)

    return gather_k


def _sc_gather(table, idx):
    V, D = table.shape
    R, K = idx.shape
    B = R * K
    CH = 64 if D > 128 else 128
    nch = B // _NW // CH
    NBUF = 4 if nch % 4 == 0 else (2 if nch % 2 == 0 else nch)
    k = _make_sc_gather(V, D, B, CH, NBUF, table.dtype.type)
    return k(table, idx.reshape(B // CH, CH)).reshape(R, K, D)


def _pad_rows(idx2d, rows_to):
    n = idx2d.shape[0]
    return jnp.pad(idx2d, ((0, rows_to - n), (0, 0)))


def _head_kernel(x_ref, w_ref, b_ref, o_ref):
    o_ref[...] = x_ref[...] @ w_ref[...] + b_ref[...]


def _head(x, w, b):
    n = x.shape[0]
    wp = jnp.pad(w, ((0, 0), (0, 128 - w.shape[1])))
    bp = jnp.pad(b, ((0, 128 - b.shape[0]),)).reshape(1, 128)
    out = pl.pallas_call(
        _head_kernel,
        out_shape=jax.ShapeDtypeStruct((n, 128), jnp.float32),
        grid=(n // 2000,),
        in_specs=[
            pl.BlockSpec((2000, x.shape[1]), lambda i: (i, 0)),
            pl.BlockSpec((x.shape[1], 128), lambda i: (0, 0)),
            pl.BlockSpec((1, 128), lambda i: (0, 0)),
        ],
        out_specs=pl.BlockSpec((2000, 128), lambda i: (i, 0)),
    )(x, wp, bp)
    return out[:, :w.shape[1]]


def _leaky(x):
    return jax.nn.leaky_relu(x, 0.1)


def _bn(x, g, b):
    m = jnp.mean(x, axis=0, keepdims=True)
    v = jnp.var(x, axis=0, keepdims=True)
    return g * (x - m) * lax.rsqrt(v + 1e-5) + b


def _kpconv(q_pts, feats, nb_pad, nq, kpts, w, sigma):
    sp = jnp.pad(q_pts, ((0, 0), (0, 13)))
    npts = _sc_gather(sp, nb_pad)[:nq, :, :3]
    rel = npts - q_pts[:, None, :]
    d = jnp.sqrt(jnp.sum((rel[:, :, None, :] - kpts[None, None, :, :]) ** 2, axis=-1) + 1e-12)
    infl = jnp.maximum(0.0, 1.0 - d / sigma)
    nf = _sc_gather(feats, nb_pad)[:nq]
    wf = jnp.einsum('nkp,nkc->npc', infl, nf)
    return jnp.einsum('npc,pcd->nd', wf, w)


def _inv_res(src_feats, nb_pad, nq, p, strided, groups=8):
    x = _leaky(_bn(src_feats @ p['wd'], p['g1'], p['b1']))
    nx = _sc_gather(x, nb_pad)[:nq]
    N, K, mid = nx.shape
    ctx = jnp.mean(nx, axis=1)
    attn = jax.nn.softmax((ctx @ p['wg']).reshape(N, K, groups), axis=1)
    nxg = nx.reshape(N, K, groups, mid // groups)
    agg = jnp.sum(attn[..., None] * nxg, axis=1).reshape(N, mid)
    out = _bn(agg @ p['wu'], p['g2'], p['b2'])
    if strided:
        sc = jnp.max(_sc_gather(src_feats, nb_pad)[:nq], axis=1)
    else:
        sc = src_feats
    if 'wsc' in p:
        sc = sc @ p['wsc']
    return _leaky(out + sc)


def _unary(x, p):
    return _leaky(_bn(x @ p['w'], p['g'], p['b']))


def _upsample(feats, ups_pad, nq, dists):
    nf = _sc_gather(feats, ups_pad)[:nq]
    w = 1.0 / (dists + 1e-6)
    w = w / jnp.sum(w, axis=1, keepdims=True)
    return jnp.sum(nf * w[..., None], axis=1)


def kernel(features, points0, points1, neighbors0, neighbors1, pools0, upsamples0, up_distances0, params):
    nb0_pad = _pad_rows(neighbors0, 10240)
    nb1_pad = _pad_rows(neighbors1, 2560)
    pools_pad = _pad_rows(pools0, 2560)
    ups_pad = _pad_rows(upsamples0, 10240)

    f = _leaky(_bn(_kpconv(points0, features, nb0_pad, 10000, params['kpts'], params['kpw'], 1.0), params['kpg'], params['kpb']))
    f = _inv_res(f, nb0_pad, 10000, params['res1'], False)
    skip = f
    f = _inv_res(f, pools_pad, 2500, params['pool1'], True)
    f = _inv_res(f, nb1_pad, 2500, params['res2'], False)
    f = _upsample(f, ups_pad, 10000, up_distances0)
    f = jnp.concatenate([f, skip], axis=1)
    f = _unary(f, params['dec1'])
    f = _unary(f, params['head1'])
    return _head(f, params['head_w'], params['head_b'])
